# baseline (speedup 1.0000x reference)
import functools

import jax
import jax.numpy as jnp
from jax import lax
from jax.experimental import pallas as pl
from jax.experimental.pallas import tpu as pltpu
from jax.experimental.pallas import tpu_sc as plsc

B, R, G, C = 4, 5000, 50, 81
NW = 32
WPB = NW // B
NRPW = R // WPB
CH = 640
NV = CH // 16
GP = 64
POS_THRESH = 0.5
LN2 = 0.6931471805599453
BIG_I = 1 << 30
RB = 4000

_mesh = plsc.VectorSubcoreMesh(
    core_axis_name="c", subcore_axis_name="s", num_cores=2, num_subcores=16)
_sc_params = pltpu.CompilerParams(needs_layout_passes=False)


def _lane():
    return lax.broadcasted_iota(jnp.int32, (16,), 0)


def _splat_f(x):
    return jnp.full((16,), x, jnp.float32)


def _splat_i(x):
    return jnp.full((16,), x, jnp.int32)


def _ln(x):
    bits = plsc.bitcast(x, jnp.int32)
    e = ((bits >> 23) & 0xFF) - 127
    m = plsc.bitcast((bits & 0x7FFFFF) | 0x3F800000, jnp.float32)
    big = m > 1.4142135623730951
    m = jnp.where(big, m * 0.5, m)
    e = e + jnp.where(big, 1, 0)
    z = (m - 1.0) / (m + 1.0)
    z2 = z * z
    p = 2.0 * z * (1.0 + z2 * (1.0 / 3.0 + z2 * (0.2 + z2 * (1.0 / 7.0 + z2 * (1.0 / 9.0)))))
    return e.astype(jnp.float32) * LN2 + p


def _window(wid):
    w8 = wid % WPB
    return jnp.minimum(w8 * 624, R - CH)


def _sc1_body(rois_h, gt_h,
              piou_h, pidx_h, lastg_h,
              raw_v, rl_v, rt_v, rr_v, rb_v, gt_v, area_v, lastg_v,
              piou_v, pidx_v):
    wid = lax.axis_index("s") * 2 + lax.axis_index("c")
    b = wid // WPB
    off0 = _window(wid)
    start = pl.multiple_of(b * (R * 4) + off0 * 4, 32)
    pltpu.sync_copy(rois_h.at[pl.ds(start, CH * 4)], raw_v)
    pltpu.sync_copy(gt_h.at[b], gt_v)
    lane = _lane()

    def init_j(j, unused):
        s = pl.ds(j * 16, 16)
        i4 = (j * 16 + lane) * 4
        rl = plsc.load_gather(raw_v, [i4])
        rt = plsc.load_gather(raw_v, [i4 + 1])
        rr = plsc.load_gather(raw_v, [i4 + 2])
        rb = plsc.load_gather(raw_v, [i4 + 3])
        rl_v[s] = rl
        rt_v[s] = rt
        rr_v[s] = rr
        rb_v[s] = rb
        area_v[s] = (rr - rl) * (rb - rt)
        lastg_v[s] = _splat_i(-1)
        return 0

    lax.fori_loop(0, NV, init_j, 0)

    def g_loop(g, unused):
        gsp0 = _splat_i(g)
        gl = plsc.load_gather(gt_v, [gsp0])
        gt0 = plsc.load_gather(gt_v, [gsp0 + GP])
        gw = plsc.load_gather(gt_v, [gsp0 + 2 * GP])
        gh = plsc.load_gather(gt_v, [gsp0 + 3 * GP])
        gr = gl + gw
        gb = gt0 + gh
        ga = gw * gh

        def j_loop(j, carry):
            bint, bun, bidx = carry
            s = pl.ds(j * 16, 16)
            iw = jnp.maximum(jnp.minimum(rr_v[s], gr) - jnp.maximum(rl_v[s], gl), 0.0)
            ih = jnp.maximum(jnp.minimum(rb_v[s], gb) - jnp.maximum(rt_v[s], gt0), 0.0)
            inter = iw * ih
            union = ga + area_v[s] - inter
            lastg_v[s] = jnp.where(inter + inter > union, g, lastg_v[s])
            ridx = off0 + j * 16 + lane
            upd = inter * bun > bint * union
            return (jnp.where(upd, inter, bint), jnp.where(upd, union, bun),
                    jnp.where(upd, ridx, bidx))

        bint, bun, bidx = lax.fori_loop(
            0, NV, j_loop, (_splat_f(-1.0), _splat_f(1.0), _splat_i(0)))
        biou = bint / bun
        m = jnp.max(biou)
        mi = jnp.min(jnp.where(biou == m, bidx, BIG_I))
        gsp = _splat_i(g)
        plsc.store_scatter(piou_v, [gsp], _splat_f(m), mask=lane == 0)
        plsc.store_scatter(pidx_v, [gsp], _splat_i(mi), mask=lane == 0)
        return 0

    lax.fori_loop(0, G, g_loop, 0)

    pltpu.sync_copy(piou_v, piou_h.at[pl.ds(wid * GP, GP)])
    pltpu.sync_copy(pidx_v, pidx_h.at[pl.ds(wid * GP, GP)])
    pltpu.sync_copy(lastg_v, lastg_h.at[pl.ds(wid * CH, CH)])


_sc1 = functools.partial(
    pl.kernel,
    out_type=[
        jax.ShapeDtypeStruct((NW * GP,), jnp.float32),
        jax.ShapeDtypeStruct((NW * GP,), jnp.int32),
        jax.ShapeDtypeStruct((NW * CH,), jnp.int32),
    ],
    mesh=_mesh,
    compiler_params=_sc_params,
    scratch_types=[
        pltpu.VMEM((4 * CH,), jnp.float32),
        pltpu.VMEM((CH,), jnp.float32),
        pltpu.VMEM((CH,), jnp.float32),
        pltpu.VMEM((CH,), jnp.float32),
        pltpu.VMEM((CH,), jnp.float32),
        pltpu.VMEM((4 * GP,), jnp.float32),
        pltpu.VMEM((CH,), jnp.float32),
        pltpu.VMEM((CH,), jnp.int32),
        pltpu.VMEM((GP,), jnp.float32),
        pltpu.VMEM((GP,), jnp.int32),
    ],
)(_sc1_body)


def _sc2_body(rois_h, loc_h, gt_h, gtl_h,
              cls_h, piou_h, pidx_h, lastg_h,
              locs_h,
              rraw_v, lraw_v, gt_v, gtl_v,
              piou_v, pidx_v, lastg_v, bestroi_v, assign_v, idx_v, pick_v,
              part_v, sem):
    wid = lax.axis_index("s") * 2 + lax.axis_index("c")
    b = wid // WPB
    off0 = _window(wid)
    lo = (wid % WPB) * NRPW
    hi = lo + NRPW
    start = pl.multiple_of(b * (R * 4) + off0 * 4, 32)
    pltpu.sync_copy(rois_h.at[pl.ds(start, CH * 4)], rraw_v)
    pltpu.sync_copy(loc_h.at[pl.ds(start, CH * 4)], lraw_v)
    pltpu.sync_copy(gt_h.at[b], gt_v)
    pltpu.sync_copy(gtl_h.at[b], gtl_v)
    pltpu.sync_copy(piou_h, piou_v)
    pltpu.sync_copy(pidx_h, pidx_v)
    pltpu.sync_copy(lastg_h.at[pl.ds(wid * CH, CH)], lastg_v)
    lane = _lane()

    wlane = b * WPB + jnp.minimum(lane, WPB - 1)

    def bg_loop(g, unused):
        gsp = _splat_i(g)
        flat = wlane * GP + gsp
        vi = plsc.load_gather(piou_v, [flat])
        xi = plsc.load_gather(pidx_v, [flat])
        vi = jnp.where(lane < WPB, vi, -2.0)
        m = jnp.max(vi)
        mi = jnp.min(jnp.where(vi == m, xi, BIG_I))
        plsc.store_scatter(bestroi_v, [gsp], _splat_i(mi), mask=lane == 0)
        return 0

    lax.fori_loop(0, G, bg_loop, 0)

    def ainit(j, unused):
        assign_v[pl.ds(j * 16, 16)] = _splat_i(-1)
        return 0

    lax.fori_loop(0, NV, ainit, 0)

    def ag_loop(g, unused):
        rv = plsc.load_gather(bestroi_v, [_splat_i(g)])
        inr = (rv >= lo) & (rv < hi)
        locc = jnp.clip(rv - off0, 0, CH - 1)
        plsc.store_scatter(assign_v, [locc], _splat_i(g),
                           mask=(lane == 0) & inr)
        return 0

    lax.fori_loop(0, G, ag_loop, 0)

    def j_loop2(j, carry):
        locacc, npacc = carry
        s = pl.ds(j * 16, 16)
        a = jnp.where(lastg_v[s] >= 0, lastg_v[s], assign_v[s])
        valid = a >= 0
        ridx = off0 + j * 16 + lane
        resp = (ridx >= lo) & (ridx < hi)
        ac = jnp.clip(a, 0, G - 1)
        lab = plsc.load_gather(gtl_v, [ac])
        clst = jnp.where(valid & resp, lab, 0)
        pidx_flat = (b * R + ridx) * C + clst
        idx_v[s] = jnp.where(resp, pidx_flat, 0)
        gl = plsc.load_gather(gt_v, [ac])
        gt0 = plsc.load_gather(gt_v, [ac + GP])
        gw = plsc.load_gather(gt_v, [ac + 2 * GP])
        gh = plsc.load_gather(gt_v, [ac + 3 * GP])
        gcx = gl + gw * 0.5
        gcy = gt0 + gh * 0.5
        i4 = (j * 16 + lane) * 4
        rl = plsc.load_gather(rraw_v, [i4])
        rt = plsc.load_gather(rraw_v, [i4 + 1])
        rr = plsc.load_gather(rraw_v, [i4 + 2])
        rb = plsc.load_gather(rraw_v, [i4 + 3])
        rw = rr - rl
        rh = rb - rt
        tx = (gcx - (rl + rr) * 0.5) / rw
        ty = (gcy - (rt + rb) * 0.5) / rh
        tw = _ln(gw / rw)
        th = _ln(gh / rh)
        pos = clst != 0
        acc = locacc
        for k, t in ((0, tx), (1, ty), (2, tw), (3, th)):
            d = plsc.load_gather(lraw_v, [i4 + k]) - t
            ad = jnp.abs(d)
            sl = jnp.where(ad < 1.0, 0.5 * d * d, ad - 0.5)
            acc = acc + jnp.where(pos, sl, 0.0)
        return acc, npacc + jnp.where(pos, 1.0, 0.0)

    locacc, npacc = lax.fori_loop(0, NV, j_loop2, (_splat_f(0.0), _splat_f(0.0)))

    descs = [
        pltpu.async_copy(cls_h.at[idx_v.at[pl.ds(k * 128, 128)]],
                         pick_v.at[pl.ds(k * 128, 128)], sem)
        for k in range(CH // 128)
    ]
    for d in descs:
        d.wait()

    def p_loop(j, pkacc):
        s = pl.ds(j * 16, 16)
        ridx = off0 + j * 16 + lane
        resp = (ridx >= lo) & (ridx < hi)
        return pkacc + jnp.where(resp, pick_v[s], 0.0)

    pkacc = lax.fori_loop(0, NV, p_loop, _splat_f(0.0))
    part_v[pl.ds(0, 16)] = locacc
    part_v[pl.ds(16, 16)] = pkacc
    part_v[pl.ds(32, 16)] = npacc
    pltpu.sync_copy(part_v, locs_h.at[wid])


_sc2 = functools.partial(
    pl.kernel,
    out_type=[
        jax.ShapeDtypeStruct((NW, 48), jnp.float32),
    ],
    mesh=_mesh,
    compiler_params=_sc_params,
    scratch_types=[
        pltpu.VMEM((4 * CH,), jnp.float32),
        pltpu.VMEM((4 * CH,), jnp.float32),
        pltpu.VMEM((4 * GP,), jnp.float32),
        pltpu.VMEM((GP,), jnp.int32),
        pltpu.VMEM((NW * GP,), jnp.float32),
        pltpu.VMEM((NW * GP,), jnp.int32),
        pltpu.VMEM((CH,), jnp.int32),
        pltpu.VMEM((GP,), jnp.int32),
        pltpu.VMEM((CH,), jnp.int32),
        pltpu.VMEM((CH,), jnp.int32),
        pltpu.VMEM((CH,), jnp.float32),
        pltpu.VMEM((48,), jnp.float32),
        pltpu.SemaphoreType.DMA,
    ],
)(_sc2_body)


def _tc_body(cls_ref, out_ref, acc_ref):
    i = pl.program_id(0)

    @pl.when(i == 0)
    def _():
        acc_ref[0] = 0.0

    cp = cls_ref[...]
    m = jnp.max(cp, axis=1, keepdims=True)
    s = jnp.sum(jnp.exp(cp - m), axis=1, keepdims=True)
    lse = m + jnp.log(s)
    acc_ref[0] = acc_ref[0] + jnp.sum(lse)

    @pl.when(i == pl.num_programs(0) - 1)
    def _():
        out_ref[0, 0] = acc_ref[0]


_tc = pl.pallas_call(
    _tc_body,
    grid=(B * R // RB,),
    in_specs=[
        pl.BlockSpec((RB, C), lambda i: (i, 0)),
    ],
    out_specs=pl.BlockSpec((1, 1), lambda i: (0, 0), memory_space=pltpu.SMEM),
    out_shape=jax.ShapeDtypeStruct((1, 1), jnp.float32),
    scratch_shapes=[pltpu.SMEM((1,), jnp.float32)],
)


def kernel(loc_p, cls_p, rois, gt_bboxes, gt_labels):
    rois2 = rois.reshape(-1)
    loc2 = loc_p.reshape(-1)
    gt_a = jnp.pad(jnp.moveaxis(gt_bboxes, 1, 2),
                   ((0, 0), (0, 0), (0, GP - G))).reshape(B, 4 * GP)
    gtl_a = jnp.pad(gt_labels.astype(jnp.int32), ((0, 0), (0, GP - G)))

    piou, pidx, lastg = _sc1(rois2, gt_a)
    (parts,) = _sc2(rois2, loc2, gt_a, gtl_a, cls_p.reshape(-1),
                    piou, pidx, lastg)
    lse_sum = _tc(cls_p).reshape(())
    p3 = parts.reshape(NW, 3, 16).sum(axis=(0, 2))
    return (p3[0] + lse_sum - p3[1]) / jnp.maximum(p3[2], 1.0)


# speedup vs baseline: 1.3280x; 1.3280x over previous
import functools

import jax
import jax.numpy as jnp
from jax import lax
from jax.experimental import pallas as pl
from jax.experimental.pallas import tpu as pltpu
from jax.experimental.pallas import tpu_sc as plsc

B, R, G, C = 4, 5000, 50, 81
RP = 5120
NW = 32
WPB = NW // B
CH = RP // WPB
NV = CH // 16
GP = 64
POS_THRESH = 0.5
LN2 = 0.6931471805599453
BIG_I = 1 << 30
RB = 4000

_mesh = plsc.VectorSubcoreMesh(
    core_axis_name="c", subcore_axis_name="s", num_cores=2, num_subcores=16)
_sc_params = pltpu.CompilerParams(needs_layout_passes=False)


def _lane():
    return lax.broadcasted_iota(jnp.int32, (16,), 0)


def _splat_f(x):
    return jnp.full((16,), x, jnp.float32)


def _splat_i(x):
    return jnp.full((16,), x, jnp.int32)


def _ln(x):
    bits = plsc.bitcast(x, jnp.int32)
    e = ((bits >> 23) & 0xFF) - 127
    m = plsc.bitcast((bits & 0x7FFFFF) | 0x3F800000, jnp.float32)
    big = m > 1.4142135623730951
    m = jnp.where(big, m * 0.5, m)
    e = e + jnp.where(big, 1, 0)
    z = (m - 1.0) / (m + 1.0)
    z2 = z * z
    p = 2.0 * z * (1.0 + z2 * (1.0 / 3.0 + z2 * (0.2 + z2 * (1.0 / 7.0 + z2 * (1.0 / 9.0)))))
    return e.astype(jnp.float32) * LN2 + p


def _sc1_body(rl_h, rt_h, rr_h, rb_h, gt_h,
              piou_h, pidx_h, lastg_h,
              rl_v, rt_v, rr_v, rb_v, gt_v, area_v, lastg_v, piou_v, pidx_v):
    wid = lax.axis_index("s") * 2 + lax.axis_index("c")
    b = wid // WPB
    lbase = (wid % WPB) * CH
    pltpu.sync_copy(rl_h.at[b, pl.ds(lbase, CH)], rl_v)
    pltpu.sync_copy(rt_h.at[b, pl.ds(lbase, CH)], rt_v)
    pltpu.sync_copy(rr_h.at[b, pl.ds(lbase, CH)], rr_v)
    pltpu.sync_copy(rb_h.at[b, pl.ds(lbase, CH)], rb_v)
    pltpu.sync_copy(gt_h.at[b], gt_v)
    lane = _lane()

    def init_j(j, unused):
        s = pl.ds(j * 16, 16)
        area_v[s] = (rr_v[s] - rl_v[s]) * (rb_v[s] - rt_v[s])
        lastg_v[s] = _splat_i(-1)
        return 0

    lax.fori_loop(0, NV, init_j, 0)

    def g_loop(g, unused):
        gsp0 = _splat_i(g)
        gl = plsc.load_gather(gt_v, [gsp0])
        gt0 = plsc.load_gather(gt_v, [gsp0 + GP])
        gw = plsc.load_gather(gt_v, [gsp0 + 2 * GP])
        gh = plsc.load_gather(gt_v, [gsp0 + 3 * GP])
        gr = gl + gw
        gb = gt0 + gh
        ga = gw * gh

        def j_loop(j, carry):
            bint, bun, bidx = carry
            s = pl.ds(j * 16, 16)
            iw = jnp.maximum(jnp.minimum(rr_v[s], gr) - jnp.maximum(rl_v[s], gl), 0.0)
            ih = jnp.maximum(jnp.minimum(rb_v[s], gb) - jnp.maximum(rt_v[s], gt0), 0.0)
            inter = iw * ih
            union = ga + area_v[s] - inter
            lastg_v[s] = jnp.where(inter + inter > union, g, lastg_v[s])
            ridx = lbase + j * 16 + lane
            upd = inter * bun > bint * union
            return (jnp.where(upd, inter, bint), jnp.where(upd, union, bun),
                    jnp.where(upd, ridx, bidx))

        bint, bun, bidx = lax.fori_loop(
            0, NV, j_loop, (_splat_f(-1.0), _splat_f(1.0), _splat_i(0)))
        biou = bint / bun
        m = jnp.max(biou)
        mi = jnp.min(jnp.where(biou == m, bidx, BIG_I))
        gsp = _splat_i(g)
        plsc.store_scatter(piou_v, [gsp], _splat_f(m), mask=lane == 0)
        plsc.store_scatter(pidx_v, [gsp], _splat_i(mi), mask=lane == 0)
        return 0

    lax.fori_loop(0, G, g_loop, 0)

    pltpu.sync_copy(piou_v, piou_h.at[pl.ds(wid * GP, GP)])
    pltpu.sync_copy(pidx_v, pidx_h.at[pl.ds(wid * GP, GP)])
    pltpu.sync_copy(lastg_v, lastg_h.at[pl.ds(wid * CH, CH)])


_sc1 = functools.partial(
    pl.kernel,
    out_type=[
        jax.ShapeDtypeStruct((NW * GP,), jnp.float32),
        jax.ShapeDtypeStruct((NW * GP,), jnp.int32),
        jax.ShapeDtypeStruct((NW * CH,), jnp.int32),
    ],
    mesh=_mesh,
    compiler_params=_sc_params,
    scratch_types=[
        pltpu.VMEM((CH,), jnp.float32),
        pltpu.VMEM((CH,), jnp.float32),
        pltpu.VMEM((CH,), jnp.float32),
        pltpu.VMEM((CH,), jnp.float32),
        pltpu.VMEM((4 * GP,), jnp.float32),
        pltpu.VMEM((CH,), jnp.float32),
        pltpu.VMEM((CH,), jnp.int32),
        pltpu.VMEM((GP,), jnp.float32),
        pltpu.VMEM((GP,), jnp.int32),
    ],
)(_sc1_body)


def _sc2_body(rl_h, rt_h, rr_h, rb_h, l0_h, l1_h, l2_h, l3_h, gt_h, gtl_h,
              cls_h, piou_h, pidx_h, lastg_h,
              locs_h,
              rl_v, rt_v, rr_v, rb_v, l0_v, l1_v, l2_v, l3_v, gt_v, gtl_v,
              piou_v, pidx_v, lastg_v, bestroi_v, assign_v, idx_v, pick_v,
              part_v, sem):
    wid = lax.axis_index("s") * 2 + lax.axis_index("c")
    b = wid // WPB
    lbase = (wid % WPB) * CH
    pltpu.sync_copy(rl_h.at[b, pl.ds(lbase, CH)], rl_v)
    pltpu.sync_copy(rt_h.at[b, pl.ds(lbase, CH)], rt_v)
    pltpu.sync_copy(rr_h.at[b, pl.ds(lbase, CH)], rr_v)
    pltpu.sync_copy(rb_h.at[b, pl.ds(lbase, CH)], rb_v)
    pltpu.sync_copy(l0_h.at[b, pl.ds(lbase, CH)], l0_v)
    pltpu.sync_copy(l1_h.at[b, pl.ds(lbase, CH)], l1_v)
    pltpu.sync_copy(l2_h.at[b, pl.ds(lbase, CH)], l2_v)
    pltpu.sync_copy(l3_h.at[b, pl.ds(lbase, CH)], l3_v)
    pltpu.sync_copy(gt_h.at[b], gt_v)
    pltpu.sync_copy(gtl_h.at[b], gtl_v)
    pltpu.sync_copy(piou_h, piou_v)
    pltpu.sync_copy(pidx_h, pidx_v)
    pltpu.sync_copy(lastg_h.at[pl.ds(wid * CH, CH)], lastg_v)
    lane = _lane()

    wlane = b * WPB + jnp.minimum(lane, WPB - 1)

    def bg_loop(g, unused):
        gsp = _splat_i(g)
        flat = wlane * GP + gsp
        vi = plsc.load_gather(piou_v, [flat])
        xi = plsc.load_gather(pidx_v, [flat])
        vi = jnp.where(lane < WPB, vi, -2.0)
        m = jnp.max(vi)
        mi = jnp.min(jnp.where(vi == m, xi, BIG_I))
        plsc.store_scatter(bestroi_v, [gsp], _splat_i(mi), mask=lane == 0)
        return 0

    lax.fori_loop(0, G, bg_loop, 0)

    def ainit(j, unused):
        assign_v[pl.ds(j * 16, 16)] = _splat_i(-1)
        return 0

    lax.fori_loop(0, NV, ainit, 0)

    def ag_loop(g, unused):
        rv = plsc.load_gather(bestroi_v, [_splat_i(g)])
        loc = rv - lbase
        inr = (loc >= 0) & (loc < CH)
        locc = jnp.clip(loc, 0, CH - 1)
        plsc.store_scatter(assign_v, [locc], _splat_i(g),
                           mask=(lane == 0) & inr)
        return 0

    lax.fori_loop(0, G, ag_loop, 0)

    def j_loop2(j, carry):
        locacc, npacc = carry
        s = pl.ds(j * 16, 16)
        a = jnp.where(lastg_v[s] >= 0, lastg_v[s], assign_v[s])
        valid = a >= 0
        gr_idx = lbase + j * 16 + lane
        realm = gr_idx < R
        ac = jnp.clip(a, 0, G - 1)
        lab = plsc.load_gather(gtl_v, [ac])
        clst = jnp.where(valid & realm, lab, 0)
        pidx_flat = (b * R + gr_idx) * C + clst
        idx_v[s] = jnp.where(realm, pidx_flat, 0)
        gl = plsc.load_gather(gt_v, [ac])
        gt0 = plsc.load_gather(gt_v, [ac + GP])
        gw = plsc.load_gather(gt_v, [ac + 2 * GP])
        gh = plsc.load_gather(gt_v, [ac + 3 * GP])
        gcx = gl + gw * 0.5
        gcy = gt0 + gh * 0.5
        rl = rl_v[s]
        rt = rt_v[s]
        rr = rr_v[s]
        rb = rb_v[s]
        rw = rr - rl
        rh = rb - rt
        tx = (gcx - (rl + rr) * 0.5) / rw
        ty = (gcy - (rt + rb) * 0.5) / rh
        tw = _ln(gw / rw)
        th = _ln(gh / rh)
        pos = clst != 0
        acc = locacc
        for lp, t in ((l0_v[s], tx), (l1_v[s], ty), (l2_v[s], tw), (l3_v[s], th)):
            d = lp - t
            ad = jnp.abs(d)
            sl = jnp.where(ad < 1.0, 0.5 * d * d, ad - 0.5)
            acc = acc + jnp.where(pos, sl, 0.0)
        return acc, npacc + jnp.where(pos, 1.0, 0.0)

    locacc, npacc = lax.fori_loop(0, NV, j_loop2, (_splat_f(0.0), _splat_f(0.0)))

    descs = [
        pltpu.async_copy(cls_h.at[idx_v.at[pl.ds(k * 128, 128)]],
                         pick_v.at[pl.ds(k * 128, 128)], sem)
        for k in range(CH // 128)
    ]
    for d in descs:
        d.wait()

    def p_loop(j, pkacc):
        s = pl.ds(j * 16, 16)
        realm = (lbase + j * 16 + lane) < R
        return pkacc + jnp.where(realm, pick_v[s], 0.0)

    pkacc = lax.fori_loop(0, NV, p_loop, _splat_f(0.0))
    part_v[pl.ds(0, 16)] = locacc
    part_v[pl.ds(16, 16)] = pkacc
    part_v[pl.ds(32, 16)] = npacc
    pltpu.sync_copy(part_v, locs_h.at[wid])


_sc2 = functools.partial(
    pl.kernel,
    out_type=[
        jax.ShapeDtypeStruct((NW, 48), jnp.float32),
    ],
    mesh=_mesh,
    compiler_params=_sc_params,
    scratch_types=[
        pltpu.VMEM((CH,), jnp.float32),
        pltpu.VMEM((CH,), jnp.float32),
        pltpu.VMEM((CH,), jnp.float32),
        pltpu.VMEM((CH,), jnp.float32),
        pltpu.VMEM((CH,), jnp.float32),
        pltpu.VMEM((CH,), jnp.float32),
        pltpu.VMEM((CH,), jnp.float32),
        pltpu.VMEM((CH,), jnp.float32),
        pltpu.VMEM((4 * GP,), jnp.float32),
        pltpu.VMEM((GP,), jnp.int32),
        pltpu.VMEM((NW * GP,), jnp.float32),
        pltpu.VMEM((NW * GP,), jnp.int32),
        pltpu.VMEM((CH,), jnp.int32),
        pltpu.VMEM((GP,), jnp.int32),
        pltpu.VMEM((CH,), jnp.int32),
        pltpu.VMEM((CH,), jnp.int32),
        pltpu.VMEM((CH,), jnp.float32),
        pltpu.VMEM((48,), jnp.float32),
        pltpu.SemaphoreType.DMA,
    ],
)(_sc2_body)


def _tc_body(cls_ref, out_ref, acc_ref):
    i = pl.program_id(0)

    @pl.when(i == 0)
    def _():
        acc_ref[0] = 0.0

    cp = cls_ref[...]
    m = jnp.max(cp, axis=1, keepdims=True)
    s = jnp.sum(jnp.exp(cp - m), axis=1, keepdims=True)
    lse = m + jnp.log(s)
    acc_ref[0] = acc_ref[0] + jnp.sum(lse)

    @pl.when(i == pl.num_programs(0) - 1)
    def _():
        out_ref[0, 0] = acc_ref[0]


_tc = pl.pallas_call(
    _tc_body,
    grid=(B * R // RB,),
    in_specs=[
        pl.BlockSpec((RB, C), lambda i: (i, 0)),
    ],
    out_specs=pl.BlockSpec((1, 1), lambda i: (0, 0), memory_space=pltpu.SMEM),
    out_shape=jax.ShapeDtypeStruct((1, 1), jnp.float32),
    scratch_shapes=[pltpu.SMEM((1,), jnp.float32)],
)


def kernel(loc_p, cls_p, rois, gt_bboxes, gt_labels):
    padr = ((0, 0), (0, RP - R))
    rl_a = jnp.pad(rois[:, :, 0], padr)
    rt_a = jnp.pad(rois[:, :, 1], padr)
    rr_a = jnp.pad(rois[:, :, 2], padr)
    rb_a = jnp.pad(rois[:, :, 3], padr)
    lp = loc_p.reshape(B, R, 4)
    l0_a = jnp.pad(lp[:, :, 0], padr)
    l1_a = jnp.pad(lp[:, :, 1], padr)
    l2_a = jnp.pad(lp[:, :, 2], padr)
    l3_a = jnp.pad(lp[:, :, 3], padr)
    gt_a = jnp.pad(jnp.moveaxis(gt_bboxes, 1, 2),
                   ((0, 0), (0, 0), (0, GP - G))).reshape(B, 4 * GP)
    gtl_a = jnp.pad(gt_labels.astype(jnp.int32), ((0, 0), (0, GP - G)))

    piou, pidx, lastg = _sc1(rl_a, rt_a, rr_a, rb_a, gt_a)
    (parts,) = _sc2(rl_a, rt_a, rr_a, rb_a, l0_a, l1_a, l2_a, l3_a,
                    gt_a, gtl_a, cls_p.reshape(-1), piou, pidx, lastg)
    lse_sum = _tc(cls_p).reshape(())
    p3 = parts.reshape(NW, 3, 16).sum(axis=(0, 2))
    return (p3[0] + lse_sum - p3[1]) / jnp.maximum(p3[2], 1.0)
